# baseline (speedup 1.0000x reference)
import functools

import jax
import jax.numpy as jnp
from jax import lax
from jax.experimental import pallas as pl
from jax.experimental.pallas import tpu as pltpu
from jax.experimental.pallas import tpu_sc as plsc

E = 8
TOP_K = 2
H = 2048
F = 1024
FS = 1024
T = 2048
BLK = 128
NB = (T * TOP_K + E * BLK) // BLK
L = NB * BLK
NW = 32
CHUNK = 128


def _router_body(x_ref, gw_ref, dest8_ref, w8_ref, bexp_ref):
    x = x_ref[...]
    gw = gw_ref[...]
    logits = jnp.dot(x, gw.T, preferred_element_type=jnp.float32)
    s = jax.nn.softmax(logits, axis=-1)
    iota8 = lax.broadcasted_iota(jnp.int32, (T, E), 1)
    m1 = jnp.max(s, axis=-1, keepdims=True)
    i1 = jnp.min(jnp.where(s == m1, iota8, E), axis=-1, keepdims=True)
    s2 = jnp.where(iota8 == i1, -jnp.inf, s)
    m2 = jnp.max(s2, axis=-1, keepdims=True)
    i2 = jnp.min(jnp.where(s2 == m2, iota8, E), axis=-1, keepdims=True)
    denom = m1 + m2
    w1 = (m1 / denom)[:, 0]
    w2n = (m2 / denom)[:, 0]
    mask = ((iota8 == i1) | (iota8 == i2)).astype(jnp.float32)

    tril = (lax.broadcasted_iota(jnp.int32, (CHUNK, CHUNK), 0)
            > lax.broadcasted_iota(jnp.int32, (CHUNK, CHUNK), 1)
            ).astype(jnp.float32)
    carry = jnp.zeros((1, E), jnp.float32)
    ranks = []
    for j in range(T // CHUNK):
        mj = mask[j * CHUNK:(j + 1) * CHUNK]
        ranks.append(jnp.dot(tril, mj, preferred_element_type=jnp.float32)
                     + carry)
        carry = carry + jnp.sum(mj, axis=0, keepdims=True)
    rank = jnp.concatenate(ranks, axis=0)
    counts = carry

    r1 = jnp.sum(jnp.where(iota8 == i1, rank, 0.0), axis=-1)
    r2 = jnp.sum(jnp.where(iota8 == i2, rank, 0.0), axis=-1)

    nb_e = (counts.astype(jnp.int32) + (BLK - 1)) // BLK
    padded = (nb_e * BLK).astype(jnp.float32)
    upper = (lax.broadcasted_iota(jnp.int32, (E, E), 0)
             < lax.broadcasted_iota(jnp.int32, (E, E), 1)).astype(jnp.float32)
    off = jnp.dot(padded, upper, preferred_element_type=jnp.float32)
    off_b = jnp.broadcast_to(off, (T, E))
    o1 = jnp.sum(jnp.where(iota8 == i1, off_b, 0.0), axis=-1)
    o2 = jnp.sum(jnp.where(iota8 == i2, off_b, 0.0), axis=-1)
    dest1 = (o1 + r1).astype(jnp.int32)
    dest2 = (o2 + r2).astype(jnp.int32)

    bend = (off + padded) / BLK
    ones_col = jnp.ones((NB, 1), jnp.float32)
    bend_rows = lax.dot_general(ones_col, bend, (((1,), (0,)), ((), ())))
    iota_b = lax.broadcasted_iota(jnp.int32, (NB, E), 0)
    bexp = jnp.sum((iota_b >= bend_rows.astype(jnp.int32)).astype(jnp.int32),
                   axis=-1)
    bexp = jnp.minimum(bexp, E - 1)

    zpad = jnp.zeros((E - 2, T), jnp.int32)
    dest8_ref[...] = jnp.concatenate(
        [dest1[None, :], dest2[None, :], zpad], axis=0)
    w8_ref[...] = jnp.concatenate(
        [w1[None, :], w2n[None, :], zpad.astype(jnp.float32)], axis=0)
    bexp_ref[...] = bexp[None, :]


def _run_router(x, gate_weight):
    return pl.pallas_call(
        _router_body,
        in_specs=[
            pl.BlockSpec((T, H), lambda: (0, 0)),
            pl.BlockSpec((E, H), lambda: (0, 0)),
        ],
        out_specs=[
            pl.BlockSpec((E, T), lambda: (0, 0)),
            pl.BlockSpec((E, T), lambda: (0, 0)),
            pl.BlockSpec((1, NB), lambda: (0, 0)),
        ],
        out_shape=[
            jax.ShapeDtypeStruct((E, T), jnp.int32),
            jax.ShapeDtypeStruct((E, T), jnp.float32),
            jax.ShapeDtypeStruct((1, NB), jnp.int32),
        ],
    )(x, gate_weight)


@functools.cache
def _sc_mesh():
    return plsc.VectorSubcoreMesh(core_axis_name="c", subcore_axis_name="s")


@functools.cache
def _build_slots_kernel():
    return pl.kernel(
        _build_slots_body,
        out_type=[jax.ShapeDtypeStruct((L,), jnp.int32),
                  jax.ShapeDtypeStruct((L,), jnp.float32)],
        mesh=_sc_mesh(),
        compiler_params=pltpu.CompilerParams(needs_layout_passes=False),
        scratch_types=[
            pltpu.VMEM((T,), jnp.int32),
            pltpu.VMEM((T,), jnp.int32),
            pltpu.VMEM((T,), jnp.float32),
            pltpu.VMEM((T,), jnp.float32),
            pltpu.VMEM((L,), jnp.int32),
            pltpu.VMEM((L,), jnp.float32),
        ],
    )


def _build_slots_body(dest8_hbm, w8_hbm, perm_hbm, slotw_hbm,
                      d0_v, d1_v, w0_v, w1_v, perm_v, sw_v):
    c = lax.axis_index("c")
    s_ = lax.axis_index("s")

    @pl.when((c == 0) & (s_ == 0))
    def _():
        pltpu.sync_copy(dest8_hbm.at[0], d0_v)
        pltpu.sync_copy(dest8_hbm.at[1], d1_v)
        pltpu.sync_copy(w8_hbm.at[0], w0_v)
        pltpu.sync_copy(w8_hbm.at[1], w1_v)

        def init(i, _):
            perm_v[pl.ds(i * 16, 16)] = jnp.zeros((16,), jnp.int32)
            sw_v[pl.ds(i * 16, 16)] = jnp.zeros((16,), jnp.float32)
            return 0

        lax.fori_loop(0, L // 16, init, 0)

        def scat(i, _):
            t = i * 16 + lax.iota(jnp.int32, 16)
            d0 = d0_v[pl.ds(i * 16, 16)]
            plsc.store_scatter(perm_v, [d0], t)
            plsc.store_scatter(sw_v, [d0], w0_v[pl.ds(i * 16, 16)])
            d1 = d1_v[pl.ds(i * 16, 16)]
            plsc.store_scatter(perm_v, [d1], t)
            plsc.store_scatter(sw_v, [d1], w1_v[pl.ds(i * 16, 16)])
            return 0

        lax.fori_loop(0, T // 16, scat, 0)
        pltpu.sync_copy(perm_v, perm_hbm)
        pltpu.sync_copy(sw_v, slotw_hbm)


_RPW = L // NW
_GC = 16
_NCH = _RPW // _GC


@functools.cache
def _gather_rows_kernel():
    return pl.kernel(
        _gather_rows_body,
        out_type=jax.ShapeDtypeStruct((L, H), jnp.float32),
        mesh=_sc_mesh(),
        compiler_params=pltpu.CompilerParams(needs_layout_passes=False),
        scratch_types=[
            pltpu.VMEM((_RPW,), jnp.int32),
            pltpu.VMEM((_GC, H), jnp.float32),
            pltpu.VMEM((_GC, H), jnp.float32),
            pltpu.SemaphoreType.DMA,
            pltpu.SemaphoreType.DMA,
        ],
    )


def _gather_rows_body(x_hbm, perm_hbm, xg_hbm, idx_v, rows_a, rows_b, s0, s1):
    c = lax.axis_index("c")
    s_ = lax.axis_index("s")
    wid = s_ * 2 + c
    base = wid * _RPW
    pltpu.sync_copy(perm_hbm.at[pl.ds(base, _RPW)], idx_v)
    bufs = (rows_a, rows_b)
    sems = (s0, s1)
    cps = [None, None]
    cps[0] = pltpu.async_copy(
        x_hbm.at[idx_v.at[pl.ds(0, _GC)]], bufs[0], sems[0])
    for j in range(1, _NCH):
        cps[j % 2] = pltpu.async_copy(
            x_hbm.at[idx_v.at[pl.ds(j * _GC, _GC)]], bufs[j % 2], sems[j % 2])
        cps[(j - 1) % 2].wait()
        pltpu.sync_copy(bufs[(j - 1) % 2],
                        xg_hbm.at[pl.ds(base + (j - 1) * _GC, _GC)])
    cps[(_NCH - 1) % 2].wait()
    pltpu.sync_copy(bufs[(_NCH - 1) % 2],
                    xg_hbm.at[pl.ds(base + (_NCH - 1) * _GC, _GC)])


def _ffn_body(bexp_ref, xg_ref, w13_ref, w2_ref, sw_ref, out_ref):
    xg = xg_ref[...].astype(jnp.bfloat16)
    gu = jnp.dot(xg, w13_ref[0], preferred_element_type=jnp.float32)
    g = gu[:, :F]
    u = gu[:, F:]
    h = (g * jax.nn.sigmoid(g) * u).astype(jnp.bfloat16)
    y = jnp.dot(h, w2_ref[0], preferred_element_type=jnp.float32)
    out_ref[...] = y * sw_ref[...]


def _run_ffn(bexp_flat, xg, w1w3, w2, slotw_2d):
    grid_spec = pltpu.PrefetchScalarGridSpec(
        num_scalar_prefetch=1,
        grid=(NB,),
        in_specs=[
            pl.BlockSpec((BLK, H), lambda i, bexp: (i, 0)),
            pl.BlockSpec((1, H, 2 * F), lambda i, bexp: (bexp[i], 0, 0)),
            pl.BlockSpec((1, F, H), lambda i, bexp: (bexp[i], 0, 0)),
            pl.BlockSpec((BLK, 1), lambda i, bexp: (i, 0)),
        ],
        out_specs=pl.BlockSpec((BLK, H), lambda i, bexp: (i, 0)),
    )
    return pl.pallas_call(
        _ffn_body,
        grid_spec=grid_spec,
        out_shape=jax.ShapeDtypeStruct((L, H), jnp.float32),
    )(bexp_flat, xg, w1w3, w2, slotw_2d)


_BTS = 256


def _shared_body(x_ref, sg_ref, su_ref, sd_ref, out_ref):
    x = x_ref[...].astype(jnp.bfloat16)
    g = jnp.dot(x, sg_ref[...].T, preferred_element_type=jnp.float32)
    u = jnp.dot(x, su_ref[...].T, preferred_element_type=jnp.float32)
    h = (g * jax.nn.sigmoid(g) * u).astype(jnp.bfloat16)
    out_ref[...] = jnp.dot(h, sd_ref[...].T, preferred_element_type=jnp.float32)


def _run_shared(x, sg, su, sd):
    return pl.pallas_call(
        _shared_body,
        grid=(T // _BTS,),
        in_specs=[
            pl.BlockSpec((_BTS, H), lambda i: (i, 0)),
            pl.BlockSpec((FS, H), lambda i: (0, 0)),
            pl.BlockSpec((FS, H), lambda i: (0, 0)),
            pl.BlockSpec((H, FS), lambda i: (0, 0)),
        ],
        out_specs=pl.BlockSpec((_BTS, H), lambda i: (i, 0)),
        out_shape=jax.ShapeDtypeStruct((T, H), jnp.float32),
    )(x, sg, su, sd)


_TPW = T // NW
_CC = 8
_NCC = _TPW // _CC


@functools.cache
def _combine_kernel():
    return pl.kernel(
        _combine_body,
        out_type=jax.ShapeDtypeStruct((T, H), jnp.float32),
        mesh=_sc_mesh(),
        compiler_params=pltpu.CompilerParams(needs_layout_passes=False),
        scratch_types=[
            pltpu.VMEM((_TPW,), jnp.int32),
            pltpu.VMEM((_TPW,), jnp.int32),
            pltpu.VMEM((_CC, H), jnp.float32),
            pltpu.VMEM((_CC, H), jnp.float32),
            pltpu.VMEM((_CC, H), jnp.float32),
            pltpu.VMEM((_CC, H), jnp.float32),
            pltpu.VMEM((_CC, H), jnp.float32),
            pltpu.VMEM((_CC, H), jnp.float32),
            pltpu.SemaphoreType.DMA,
            pltpu.SemaphoreType.DMA,
        ],
    )


def _combine_body(yg_hbm, dest8_hbm, shared_hbm, out_hbm,
                  i0_v, i1_v, r0a, r0b, r1a, r1b, sha, shb, s0, s1):
    c = lax.axis_index("c")
    s_ = lax.axis_index("s")
    wid = s_ * 2 + c
    base = wid * _TPW
    pltpu.sync_copy(dest8_hbm.at[0, pl.ds(base, _TPW)], i0_v)
    pltpu.sync_copy(dest8_hbm.at[1, pl.ds(base, _TPW)], i1_v)
    r0s, r1s, shs, sems = (r0a, r0b), (r1a, r1b), (sha, shb), (s0, s1)

    def issue(j, si):
        sl = pl.ds(j * _CC, _CC)
        return (
            pltpu.async_copy(yg_hbm.at[i0_v.at[sl]], r0s[si], sems[si]),
            pltpu.async_copy(yg_hbm.at[i1_v.at[sl]], r1s[si], sems[si]),
            pltpu.async_copy(shared_hbm.at[pl.ds(base + j * _CC, _CC)],
                             shs[si], sems[si]),
        )

    def drain_compute(j, si):
        r0, r1, sh = r0s[si], r1s[si], shs[si]

        def row_loop(r, _):
            def col_loop(k, _):
                cs = pl.ds(k * 16, 16)
                r0[r, cs] = r0[r, cs] + r1[r, cs] + sh[r, cs]
                return 0
            lax.fori_loop(0, H // 16, col_loop, 0)
            return 0

        lax.fori_loop(0, _CC, row_loop, 0)
        pltpu.sync_copy(r0, out_hbm.at[pl.ds(base + j * _CC, _CC)])

    pend = [None, None]
    pend[0] = issue(0, 0)
    for j in range(1, _NCC):
        pend[j % 2] = issue(j, j % 2)
        for cp in pend[(j - 1) % 2]:
            cp.wait()
        drain_compute(j - 1, (j - 1) % 2)
    for cp in pend[(_NCC - 1) % 2]:
        cp.wait()
    drain_compute(_NCC - 1, (_NCC - 1) % 2)


@jax.jit
def kernel(hidden_states, gate_weight, w1w3, w2, shared_gate_w, shared_up_w,
           shared_down_w):
    x = hidden_states.reshape(-1, H)

    dest8, w8, bexp = _run_router(x, gate_weight)
    perm, slotw = _build_slots_kernel()(dest8, w8)
    xg = _gather_rows_kernel()(x, perm)
    yg = _run_ffn(bexp.reshape(NB), xg, w1w3.astype(jnp.bfloat16),
                  w2.astype(jnp.bfloat16), slotw.reshape(L, 1))
    shared = _run_shared(x, shared_gate_w.astype(jnp.bfloat16),
                         shared_up_w.astype(jnp.bfloat16),
                         shared_down_w.astype(jnp.bfloat16))
    out = _combine_kernel()(yg, dest8, shared)
    return out.reshape(1, T, H)


# speedup vs baseline: 1.2433x; 1.2433x over previous
import functools

import jax
import jax.numpy as jnp
from jax import lax
from jax.experimental import pallas as pl
from jax.experimental.pallas import tpu as pltpu
from jax.experimental.pallas import tpu_sc as plsc

E = 8
TOP_K = 2
H = 2048
F = 1024
FS = 1024
T = 2048
BLK = 128
NB = (T * TOP_K + E * BLK) // BLK
L = NB * BLK
NW = 32
CHUNK = 128


def _router_body(x_ref, gw_ref, dest8_ref, w8_ref, bexp_ref):
    x = x_ref[...]
    gw = gw_ref[...]
    logits = jnp.dot(x, gw.T, preferred_element_type=jnp.float32)
    s = jax.nn.softmax(logits, axis=-1)
    iota8 = lax.broadcasted_iota(jnp.int32, (T, E), 1)
    m1 = jnp.max(s, axis=-1, keepdims=True)
    i1 = jnp.min(jnp.where(s == m1, iota8, E), axis=-1, keepdims=True)
    s2 = jnp.where(iota8 == i1, -jnp.inf, s)
    m2 = jnp.max(s2, axis=-1, keepdims=True)
    i2 = jnp.min(jnp.where(s2 == m2, iota8, E), axis=-1, keepdims=True)
    denom = m1 + m2
    w1 = (m1 / denom)[:, 0]
    w2n = (m2 / denom)[:, 0]
    mask = ((iota8 == i1) | (iota8 == i2)).astype(jnp.float32)

    tril = (lax.broadcasted_iota(jnp.int32, (CHUNK, CHUNK), 0)
            > lax.broadcasted_iota(jnp.int32, (CHUNK, CHUNK), 1)
            ).astype(jnp.float32)
    carry = jnp.zeros((1, E), jnp.float32)
    ranks = []
    for j in range(T // CHUNK):
        mj = mask[j * CHUNK:(j + 1) * CHUNK]
        ranks.append(jnp.dot(tril, mj, preferred_element_type=jnp.float32)
                     + carry)
        carry = carry + jnp.sum(mj, axis=0, keepdims=True)
    rank = jnp.concatenate(ranks, axis=0)
    counts = carry

    r1 = jnp.sum(jnp.where(iota8 == i1, rank, 0.0), axis=-1)
    r2 = jnp.sum(jnp.where(iota8 == i2, rank, 0.0), axis=-1)

    nb_e = (counts.astype(jnp.int32) + (BLK - 1)) // BLK
    padded = (nb_e * BLK).astype(jnp.float32)
    upper = (lax.broadcasted_iota(jnp.int32, (E, E), 0)
             < lax.broadcasted_iota(jnp.int32, (E, E), 1)).astype(jnp.float32)
    off = jnp.dot(padded, upper, preferred_element_type=jnp.float32)
    off_b = jnp.broadcast_to(off, (T, E))
    o1 = jnp.sum(jnp.where(iota8 == i1, off_b, 0.0), axis=-1)
    o2 = jnp.sum(jnp.where(iota8 == i2, off_b, 0.0), axis=-1)
    dest1 = (o1 + r1).astype(jnp.int32)
    dest2 = (o2 + r2).astype(jnp.int32)

    bend = (off + padded) / BLK
    ones_col = jnp.ones((NB, 1), jnp.float32)
    bend_rows = lax.dot_general(ones_col, bend, (((1,), (0,)), ((), ())))
    iota_b = lax.broadcasted_iota(jnp.int32, (NB, E), 0)
    bexp = jnp.sum((iota_b >= bend_rows.astype(jnp.int32)).astype(jnp.int32),
                   axis=-1)
    bexp = jnp.minimum(bexp, E - 1)

    zpad = jnp.zeros((E - 2, T), jnp.int32)
    dest8_ref[...] = jnp.concatenate(
        [dest1[None, :], dest2[None, :], zpad], axis=0)
    w8_ref[...] = jnp.concatenate(
        [w1[None, :], w2n[None, :], zpad.astype(jnp.float32)], axis=0)
    bexp_ref[...] = bexp[None, :]


def _run_router(x, gate_weight):
    return pl.pallas_call(
        _router_body,
        in_specs=[
            pl.BlockSpec((T, H), lambda: (0, 0)),
            pl.BlockSpec((E, H), lambda: (0, 0)),
        ],
        out_specs=[
            pl.BlockSpec((E, T), lambda: (0, 0)),
            pl.BlockSpec((E, T), lambda: (0, 0)),
            pl.BlockSpec((1, NB), lambda: (0, 0)),
        ],
        out_shape=[
            jax.ShapeDtypeStruct((E, T), jnp.int32),
            jax.ShapeDtypeStruct((E, T), jnp.float32),
            jax.ShapeDtypeStruct((1, NB), jnp.int32),
        ],
    )(x, gate_weight)


@functools.cache
def _sc_mesh():
    return plsc.VectorSubcoreMesh(core_axis_name="c", subcore_axis_name="s")


@functools.cache
def _build_slots_kernel():
    return pl.kernel(
        _build_slots_body,
        out_type=[jax.ShapeDtypeStruct((L,), jnp.int32),
                  jax.ShapeDtypeStruct((L,), jnp.float32)],
        mesh=_sc_mesh(),
        compiler_params=pltpu.CompilerParams(needs_layout_passes=False),
        scratch_types=[
            pltpu.VMEM((T,), jnp.int32),
            pltpu.VMEM((T,), jnp.int32),
            pltpu.VMEM((T,), jnp.float32),
            pltpu.VMEM((T,), jnp.float32),
            pltpu.VMEM((L,), jnp.int32),
            pltpu.VMEM((L,), jnp.float32),
        ],
    )


def _build_slots_body(dest8_hbm, w8_hbm, perm_hbm, slotw_hbm,
                      d0_v, d1_v, w0_v, w1_v, perm_v, sw_v):
    c = lax.axis_index("c")
    s_ = lax.axis_index("s")

    @pl.when((c == 0) & (s_ == 0))
    def _():
        pltpu.sync_copy(dest8_hbm.at[0], d0_v)
        pltpu.sync_copy(dest8_hbm.at[1], d1_v)
        pltpu.sync_copy(w8_hbm.at[0], w0_v)
        pltpu.sync_copy(w8_hbm.at[1], w1_v)

        def init(i, _):
            perm_v[pl.ds(i * 16, 16)] = jnp.zeros((16,), jnp.int32)
            sw_v[pl.ds(i * 16, 16)] = jnp.zeros((16,), jnp.float32)
            return 0

        lax.fori_loop(0, L // 16, init, 0)

        def scat(i, _):
            t = i * 16 + lax.iota(jnp.int32, 16)
            d0 = d0_v[pl.ds(i * 16, 16)]
            plsc.store_scatter(perm_v, [d0], t)
            plsc.store_scatter(sw_v, [d0], w0_v[pl.ds(i * 16, 16)])
            d1 = d1_v[pl.ds(i * 16, 16)]
            plsc.store_scatter(perm_v, [d1], t)
            plsc.store_scatter(sw_v, [d1], w1_v[pl.ds(i * 16, 16)])
            return 0

        lax.fori_loop(0, T // 16, scat, 0)
        pltpu.sync_copy(perm_v, perm_hbm)
        pltpu.sync_copy(sw_v, slotw_hbm)


_RPW = L // NW
_GC = 40


@functools.cache
def _gather_rows_kernel():
    return pl.kernel(
        _gather_rows_body,
        out_type=jax.ShapeDtypeStruct((L, H), jnp.float32),
        mesh=_sc_mesh(),
        compiler_params=pltpu.CompilerParams(needs_layout_passes=False),
        scratch_types=[
            pltpu.VMEM((_RPW,), jnp.int32),
            pltpu.VMEM((_GC, H), jnp.float32),
            pltpu.SemaphoreType.DMA,
        ],
    )


def _gather_rows_body(x_hbm, perm_hbm, xg_hbm, idx_v, rows_v, sem):
    c = lax.axis_index("c")
    s_ = lax.axis_index("s")
    wid = s_ * 2 + c
    base = wid * _RPW
    pltpu.sync_copy(perm_hbm.at[pl.ds(base, _RPW)], idx_v)
    for j in range(_RPW // _GC):
        pltpu.async_copy(
            x_hbm.at[idx_v.at[pl.ds(j * _GC, _GC)]], rows_v, sem).wait()
        pltpu.sync_copy(rows_v, xg_hbm.at[pl.ds(base + j * _GC, _GC)])


def _ffn_body(bexp_ref, xg_ref, w13_ref, w2_ref, sw_ref, out_ref):
    xg = xg_ref[...]
    gu = jnp.dot(xg, w13_ref[0], preferred_element_type=jnp.float32)
    g = gu[:, :F]
    u = gu[:, F:]
    h = g * jax.nn.sigmoid(g) * u
    y = jnp.dot(h, w2_ref[0], preferred_element_type=jnp.float32)
    out_ref[...] = y * sw_ref[...]


def _run_ffn(bexp_flat, xg, w1w3, w2, slotw_2d):
    grid_spec = pltpu.PrefetchScalarGridSpec(
        num_scalar_prefetch=1,
        grid=(NB,),
        in_specs=[
            pl.BlockSpec((BLK, H), lambda i, bexp: (i, 0)),
            pl.BlockSpec((1, H, 2 * F), lambda i, bexp: (bexp[i], 0, 0)),
            pl.BlockSpec((1, F, H), lambda i, bexp: (bexp[i], 0, 0)),
            pl.BlockSpec((BLK, 1), lambda i, bexp: (i, 0)),
        ],
        out_specs=pl.BlockSpec((BLK, H), lambda i, bexp: (i, 0)),
    )
    return pl.pallas_call(
        _ffn_body,
        grid_spec=grid_spec,
        out_shape=jax.ShapeDtypeStruct((L, H), jnp.float32),
    )(bexp_flat, xg, w1w3, w2, slotw_2d)


_BTS = 256


def _shared_body(x_ref, sg_ref, su_ref, sd_ref, out_ref):
    x = x_ref[...]
    g = jnp.dot(x, sg_ref[...].T, preferred_element_type=jnp.float32)
    u = jnp.dot(x, su_ref[...].T, preferred_element_type=jnp.float32)
    h = g * jax.nn.sigmoid(g) * u
    out_ref[...] = jnp.dot(h, sd_ref[...].T, preferred_element_type=jnp.float32)


def _run_shared(x, sg, su, sd):
    return pl.pallas_call(
        _shared_body,
        grid=(T // _BTS,),
        in_specs=[
            pl.BlockSpec((_BTS, H), lambda i: (i, 0)),
            pl.BlockSpec((FS, H), lambda i: (0, 0)),
            pl.BlockSpec((FS, H), lambda i: (0, 0)),
            pl.BlockSpec((H, FS), lambda i: (0, 0)),
        ],
        out_specs=pl.BlockSpec((_BTS, H), lambda i: (i, 0)),
        out_shape=jax.ShapeDtypeStruct((T, H), jnp.float32),
    )(x, sg, su, sd)


_TPW = T // NW
_CC = 8
_NCC = _TPW // _CC


@functools.cache
def _combine_kernel():
    return pl.kernel(
        _combine_body,
        out_type=jax.ShapeDtypeStruct((T, H), jnp.float32),
        mesh=_sc_mesh(),
        compiler_params=pltpu.CompilerParams(needs_layout_passes=False),
        scratch_types=[
            pltpu.VMEM((_TPW,), jnp.int32),
            pltpu.VMEM((_TPW,), jnp.int32),
            pltpu.VMEM((_CC, H), jnp.float32),
            pltpu.VMEM((_CC, H), jnp.float32),
            pltpu.VMEM((_CC, H), jnp.float32),
            pltpu.VMEM((_CC, H), jnp.float32),
            pltpu.VMEM((_CC, H), jnp.float32),
            pltpu.VMEM((_CC, H), jnp.float32),
            pltpu.SemaphoreType.DMA,
            pltpu.SemaphoreType.DMA,
        ],
    )


def _combine_body(yg_hbm, dest8_hbm, shared_hbm, out_hbm,
                  i0_v, i1_v, r0a, r0b, r1a, r1b, sha, shb, s0, s1):
    c = lax.axis_index("c")
    s_ = lax.axis_index("s")
    wid = s_ * 2 + c
    base = wid * _TPW
    pltpu.sync_copy(dest8_hbm.at[0, pl.ds(base, _TPW)], i0_v)
    pltpu.sync_copy(dest8_hbm.at[1, pl.ds(base, _TPW)], i1_v)
    r0s, r1s, shs, sems = (r0a, r0b), (r1a, r1b), (sha, shb), (s0, s1)

    def issue(j, si):
        sl = pl.ds(j * _CC, _CC)
        return (
            pltpu.async_copy(yg_hbm.at[i0_v.at[sl]], r0s[si], sems[si]),
            pltpu.async_copy(yg_hbm.at[i1_v.at[sl]], r1s[si], sems[si]),
            pltpu.async_copy(shared_hbm.at[pl.ds(base + j * _CC, _CC)],
                             shs[si], sems[si]),
        )

    def drain_compute(j, si):
        r0, r1, sh = r0s[si], r1s[si], shs[si]

        def row_loop(r, _):
            def col_loop(k, _):
                cs = pl.ds(k * 16, 16)
                r0[r, cs] = r0[r, cs] + r1[r, cs] + sh[r, cs]
                return 0
            lax.fori_loop(0, H // 16, col_loop, 0)
            return 0

        lax.fori_loop(0, _CC, row_loop, 0)
        pltpu.sync_copy(r0, out_hbm.at[pl.ds(base + j * _CC, _CC)])

    pend = [None, None]
    pend[0] = issue(0, 0)
    for j in range(1, _NCC):
        pend[j % 2] = issue(j, j % 2)
        for cp in pend[(j - 1) % 2]:
            cp.wait()
        drain_compute(j - 1, (j - 1) % 2)
    for cp in pend[(_NCC - 1) % 2]:
        cp.wait()
    drain_compute(_NCC - 1, (_NCC - 1) % 2)


@jax.jit
def kernel(hidden_states, gate_weight, w1w3, w2, shared_gate_w, shared_up_w,
           shared_down_w):
    x = hidden_states.reshape(-1, H)

    dest8, w8, bexp = _run_router(x, gate_weight)
    perm, slotw = _build_slots_kernel()(dest8, w8)
    xg = _gather_rows_kernel()(x, perm)
    yg = _run_ffn(bexp.reshape(NB), xg, w1w3, w2, slotw.reshape(L, 1))
    shared = _run_shared(x, shared_gate_w, shared_up_w, shared_down_w)
    out = _combine_kernel()(yg, dest8, shared)
    return out.reshape(1, T, H)
